# baseline (speedup 1.0000x reference)
import functools

import jax
import jax.numpy as jnp
from jax import lax
from jax.experimental import pallas as pl
from jax.experimental.pallas import tpu as pltpu
from jax.experimental.pallas import tpu_sc as plsc

GX, GY, GZ, D = 64, 64, 64, 32
V = GX * GY * GZ
N = 262144

NC, NS, L = 2, 16, 16
NW = NC * NS
BPW = N // NW
C = 2048
NCH = BPW // C

_mesh = plsc.VectorSubcoreMesh(core_axis_name="c", subcore_axis_name="s")


@functools.partial(
    pl.kernel,
    mesh=_mesh,
    out_type=jax.ShapeDtypeStruct((N, D), jnp.float32),
    compiler_params=pltpu.CompilerParams(
        needs_layout_passes=False, use_tc_tiling_on_sc=False
    ),
    scratch_types=[
        pltpu.VMEM((C * 3,), jnp.int32),
        pltpu.VMEM((C,), jnp.int32),
        pltpu.VMEM((C, D), jnp.float32),
        pltpu.SemaphoreType.DMA,
    ],
)
def _sc_gather(table_hbm, idx_hbm, out_hbm, idx_v, lin_v, rows_v, sem):
    wid = lax.axis_index("s") * NC + lax.axis_index("c")
    base = wid * BPW

    def chunk(g, carry):
        off = base + g * C
        pltpu.sync_copy(idx_hbm.at[pl.ds(off * 3, C * 3)], idx_v)

        lane3 = lax.iota(jnp.int32, L) * 3

        def linearize(i, carry2):
            p = i * (3 * L) + lane3
            x = plsc.load_gather(idx_v, [p])
            y = plsc.load_gather(idx_v, [p + 1])
            z = plsc.load_gather(idx_v, [p + 2])
            lin_v[pl.ds(i * L, L)] = (x << 12) + (y << 6) + z
            return carry2

        lax.fori_loop(0, C // L, linearize, 0, unroll=4)

        pltpu.async_copy(table_hbm.at[lin_v], rows_v, sem).wait()
        pltpu.sync_copy(rows_v, out_hbm.at[pl.ds(off, C)])
        return carry

    lax.fori_loop(0, NCH, chunk, 0)


def kernel(in_tensor, in_index):
    table = in_tensor.reshape(V, D)
    idx = in_index.astype(jnp.int32).reshape(N * 3)
    return _sc_gather(table, idx)


# speedup vs baseline: 1.4818x; 1.4818x over previous
import functools

import jax
import jax.numpy as jnp
from jax import lax
from jax.experimental import pallas as pl
from jax.experimental.pallas import tpu as pltpu
from jax.experimental.pallas import tpu_sc as plsc

GX, GY, GZ, D = 64, 64, 64, 32
V = GX * GY * GZ
N = 262144

NC, NS, L = 2, 16, 16
NW = NC * NS
BPW = N // NW
C = 2048
NCH = BPW // C

_mesh = plsc.VectorSubcoreMesh(core_axis_name="c", subcore_axis_name="s")


@functools.partial(
    pl.kernel,
    mesh=_mesh,
    out_type=jax.ShapeDtypeStruct((N, D), jnp.float32),
    compiler_params=pltpu.CompilerParams(
        needs_layout_passes=False, use_tc_tiling_on_sc=False
    ),
    scratch_types=[
        pltpu.VMEM((C * 3,), jnp.int32),
        pltpu.VMEM((C,), jnp.int32),
        pltpu.VMEM((C, D), jnp.float32),
        pltpu.SemaphoreType.DMA,
    ],
)
def _sc_gather(table_hbm, idx_hbm, out_hbm, idx_v, lin_v, rows_v, sem):
    wid = lax.axis_index("s") * NC + lax.axis_index("c")
    base = wid * BPW

    def chunk(g, carry):
        off = base + g * C
        pltpu.sync_copy(idx_hbm.at[pl.ds(off, C)], idx_v.at[pl.ds(0, C)])
        pltpu.sync_copy(idx_hbm.at[pl.ds(N + off, C)], idx_v.at[pl.ds(C, C)])
        pltpu.sync_copy(idx_hbm.at[pl.ds(2 * N + off, C)], idx_v.at[pl.ds(2 * C, C)])

        def linearize(i, carry2):
            s = pl.ds(i * L, L)
            x = idx_v[pl.ds(i * L, L)]
            y = idx_v[pl.ds(C + i * L, L)]
            z = idx_v[pl.ds(2 * C + i * L, L)]
            lin_v[s] = (x << 12) + (y << 6) + z
            return carry2

        lax.fori_loop(0, C // L, linearize, 0, unroll=4)

        pltpu.async_copy(table_hbm.at[lin_v], rows_v, sem).wait()
        pltpu.sync_copy(rows_v, out_hbm.at[pl.ds(off, C)])
        return carry

    lax.fori_loop(0, NCH, chunk, 0)


def kernel(in_tensor, in_index):
    table = in_tensor.reshape(V, D)
    idx = in_index.astype(jnp.int32).T.reshape(3 * N)
    return _sc_gather(table, idx)
